# baseline (speedup 1.0000x reference)
import jax
import jax.numpy as jnp
from jax import lax
from jax.experimental import pallas as pl
from jax.experimental.pallas import tpu as pltpu
from jax.experimental.pallas import tpu_sc as plsc

B = 16384
EMB_DIM = 32
BERT_DIM = 768
NC = 2
NS = 16
L = 16
NW = NC * NS
BPW = B // NW
C = 32
NCH = BPW // C


def _diags():
    iot = lax.iota(jnp.int32, L)
    return [jnp.where(iot + k >= L, iot + k - L, iot + k) for k in range(L)]


def _text_body(uidf_h, iidf_h, ut_h, it_h, ub_h, ib_h, b16_h,
               out_h, uo_h, io_h,
               uidf_v, iidf_v, uid_v, iid_v, utb, itb, ubb, ibb,
               outb, b16_v, sem0, sem1):
    wid = lax.axis_index("s") * NC + lax.axis_index("c")
    base = wid * BPW
    pltpu.sync_copy(uidf_h.at[pl.ds(base, BPW)], uidf_v)
    pltpu.sync_copy(iidf_h.at[pl.ds(base, BPW)], iidf_v)
    for k in range(BPW // L):
        uid_v[pl.ds(k * L, L)] = plsc.bitcast(uidf_v[pl.ds(k * L, L)],
                                              jnp.int32)
        iid_v[pl.ds(k * L, L)] = plsc.bitcast(iidf_v[pl.ds(k * L, L)],
                                              jnp.int32)
    pltpu.sync_copy(b16_h, b16_v)
    sems = (sem0, sem1)
    diags = _diags()
    iot = lax.iota(jnp.int32, L)

    def copies(c, s):
        return (
            pltpu.make_async_copy(ut_h.at[uid_v.at[pl.ds(c * C, C)]],
                                  utb.at[s], sems[s]),
            pltpu.make_async_copy(it_h.at[iid_v.at[pl.ds(c * C, C)]],
                                  itb.at[s], sems[s]),
            pltpu.make_async_copy(ub_h.at[uid_v.at[pl.ds(c * C, C)]],
                                  ubb.at[s], sems[s]),
            pltpu.make_async_copy(ib_h.at[iid_v.at[pl.ds(c * C, C)]],
                                  ibb.at[s], sems[s]),
        )

    def compute(c, s):
        ut = utb.at[s]
        it = itb.at[s]
        for g in range(C // L):
            rows = iot + g * L

            def body(jb, accs):
                accs = list(accs)
                cb = jnp.full((L,), jb * L, jnp.int32)
                for k in range(L):
                    col = cb + diags[k]
                    u = plsc.load_gather(ut, [rows, col])
                    v = plsc.load_gather(it, [rows, col])
                    accs[k & 3] = accs[k & 3] + u * v
                return tuple(accs)

            zero = jnp.zeros((L,), jnp.float32)
            accs = lax.fori_loop(0, BERT_DIM // L, body,
                                 (zero, zero, zero, zero))
            a0, a1, a2, a3 = accs
            bia = ubb.at[s][pl.ds(g * L, L)] + ibb.at[s][pl.ds(g * L, L)]
            outb[pl.ds(c * C + g * L, L)] = ((a0 + a1) + (a2 + a3)
                                             + bia + b16_v[...])

    for d in copies(0, 0):
        d.start()

    def chunk_pair(i, carry):
        c = 2 * i
        for d in copies(c + 1, 1):
            d.start()
        for d in copies(c, 0):
            d.wait()
        compute(c, 0)

        @pl.when(i < NCH // 2 - 1)
        def _():
            for d in copies(c + 2, 0):
                d.start()

        for d in copies(c + 1, 1):
            d.wait()
        compute(c + 1, 1)
        return carry

    lax.fori_loop(0, NCH // 2, chunk_pair, 0)

    pltpu.sync_copy(outb, out_h.at[pl.ds(base, BPW)])
    pltpu.sync_copy(uidf_v, uo_h.at[pl.ds(base, BPW)])
    pltpu.sync_copy(iidf_v, io_h.at[pl.ds(base, BPW)])


def _emb_body(uidf_h, iidf_h, ue_h, ie_h, out_h,
              uidf_v, iidf_v, uid_v, iid_v, ueb, ieb, outb, sem0):
    wid = lax.axis_index("s") * NC + lax.axis_index("c")
    base = wid * BPW
    pltpu.sync_copy(uidf_h.at[pl.ds(base, BPW)], uidf_v)
    pltpu.sync_copy(iidf_h.at[pl.ds(base, BPW)], iidf_v)
    for k in range(BPW // L):
        uid_v[pl.ds(k * L, L)] = plsc.bitcast(uidf_v[pl.ds(k * L, L)],
                                              jnp.int32)
        iid_v[pl.ds(k * L, L)] = plsc.bitcast(iidf_v[pl.ds(k * L, L)],
                                              jnp.int32)
    diags = _diags()
    iot = lax.iota(jnp.int32, L)

    ds = (
        pltpu.make_async_copy(ue_h.at[uid_v], ueb, sem0),
        pltpu.make_async_copy(ie_h.at[iid_v], ieb, sem0),
    )
    for d in ds:
        d.start()
    for d in ds:
        d.wait()

    def group(g, carry):
        rows = iot + g * L
        acc = jnp.zeros((L,), jnp.float32)
        for jb in range(EMB_DIM // L):
            cb = jnp.full((L,), jb * L, jnp.int32)
            for k in range(L):
                col = cb + diags[k]
                u = plsc.load_gather(ueb, [rows, col])
                v = plsc.load_gather(ieb, [rows, col])
                acc = acc + u * v
        outb[pl.ds(g * L, L)] = acc
        return carry

    lax.fori_loop(0, BPW // L, group, 0)

    pltpu.sync_copy(outb, out_h.at[pl.ds(base, BPW)])


def kernel(user_ids, item_ids, user_emb_w, item_emb_w, user_text_w,
           item_text_w, user_bias, item_bias, bias):
    mesh = plsc.VectorSubcoreMesh(core_axis_name="c", subcore_axis_name="s",
                                  num_cores=NC, num_subcores=NS)
    out_t = jax.ShapeDtypeStruct((B,), jnp.float32)
    id_t = jax.ShapeDtypeStruct((B,), jnp.float32)

    text_run = pl.kernel(
        _text_body,
        out_type=(out_t, id_t, id_t),
        mesh=mesh,
        compiler_params=pltpu.CompilerParams(use_tc_tiling_on_sc=True,
                                             needs_layout_passes=False),
        scratch_types=[
            pltpu.VMEM((BPW,), jnp.float32),
            pltpu.VMEM((BPW,), jnp.float32),
            pltpu.VMEM((BPW,), jnp.int32),
            pltpu.VMEM((BPW,), jnp.int32),
            pltpu.VMEM((2, C, BERT_DIM), jnp.float32),
            pltpu.VMEM((2, C, BERT_DIM), jnp.float32),
            pltpu.VMEM((2, C), jnp.float32),
            pltpu.VMEM((2, C), jnp.float32),
            pltpu.VMEM((BPW,), jnp.float32),
            pltpu.VMEM((L,), jnp.float32),
            pltpu.SemaphoreType.DMA,
            pltpu.SemaphoreType.DMA,
        ],
    )
    emb_run = pl.kernel(
        _emb_body,
        out_type=out_t,
        mesh=mesh,
        compiler_params=pltpu.CompilerParams(use_tc_tiling_on_sc=False,
                                             needs_layout_passes=False),
        scratch_types=[
            pltpu.VMEM((BPW,), jnp.float32),
            pltpu.VMEM((BPW,), jnp.float32),
            pltpu.VMEM((BPW,), jnp.int32),
            pltpu.VMEM((BPW,), jnp.int32),
            pltpu.VMEM((BPW, EMB_DIM), jnp.float32),
            pltpu.VMEM((BPW, EMB_DIM), jnp.float32),
            pltpu.VMEM((BPW,), jnp.float32),
            pltpu.SemaphoreType.DMA,
        ],
    )
    bias16 = jnp.broadcast_to(bias, (L,))
    uidf = lax.bitcast_convert_type(user_ids, jnp.float32)
    iidf = lax.bitcast_convert_type(item_ids, jnp.float32)
    out_text, uo, io = text_run(uidf, iidf, user_text_w,
                                item_text_w, user_bias, item_bias, bias16)
    out_emb = emb_run(uo, io, user_emb_w, item_emb_w)
    return (out_text + out_emb)[:, None]


# speedup vs baseline: 1.0101x; 1.0101x over previous
import jax
import jax.numpy as jnp
from jax import lax
from jax.experimental import pallas as pl
from jax.experimental.pallas import tpu as pltpu
from jax.experimental.pallas import tpu_sc as plsc

B = 16384
EMB_DIM = 32
BERT_DIM = 768
NC = 2
NS = 16
L = 16
NW = NC * NS
BPW = B // NW
C = 32
NCH = BPW // C


def _diags():
    iot = lax.iota(jnp.int32, L)
    return [jnp.where(iot + k >= L, iot + k - L, iot + k) for k in range(L)]


def _text_body(uid_h, iid_h, ut_h, it_h, ub_h, ib_h, b16_h,
               out_h, uo_h, io_h,
               uid_v, iid_v, utb, itb, ubb, ibb, outb, b16_v, sem0, sem1):
    wid = lax.axis_index("s") * NC + lax.axis_index("c")
    base = wid * BPW
    pltpu.sync_copy(uid_h.at[pl.ds(base, BPW)], uid_v)
    pltpu.sync_copy(iid_h.at[pl.ds(base, BPW)], iid_v)
    pltpu.sync_copy(b16_h, b16_v)
    sems = (sem0, sem1)
    diags = _diags()
    iot = lax.iota(jnp.int32, L)

    def copies(c, s):
        return (
            pltpu.make_async_copy(ut_h.at[uid_v.at[pl.ds(c * C, C)]],
                                  utb.at[s], sems[s]),
            pltpu.make_async_copy(it_h.at[iid_v.at[pl.ds(c * C, C)]],
                                  itb.at[s], sems[s]),
            pltpu.make_async_copy(ub_h.at[uid_v.at[pl.ds(c * C, C)]],
                                  ubb.at[s], sems[s]),
            pltpu.make_async_copy(ib_h.at[iid_v.at[pl.ds(c * C, C)]],
                                  ibb.at[s], sems[s]),
        )

    def compute(c, s):
        ut = utb.at[s]
        it = itb.at[s]
        for g in range(C // L):
            rows = iot + g * L

            def body(jb, accs):
                accs = list(accs)
                cb = jnp.full((L,), jb * L, jnp.int32)
                for k in range(L):
                    col = cb + diags[k]
                    u = plsc.load_gather(ut, [rows, col])
                    v = plsc.load_gather(it, [rows, col])
                    accs[k & 3] = accs[k & 3] + u * v
                return tuple(accs)

            zero = jnp.zeros((L,), jnp.float32)
            accs = lax.fori_loop(0, BERT_DIM // L, body,
                                 (zero, zero, zero, zero))
            a0, a1, a2, a3 = accs
            bia = ubb.at[s][pl.ds(g * L, L)] + ibb.at[s][pl.ds(g * L, L)]
            outb[pl.ds(c * C + g * L, L)] = ((a0 + a1) + (a2 + a3)
                                             + bia + b16_v[...])

    for d in copies(0, 0):
        d.start()

    def chunk_pair(i, carry):
        c = 2 * i
        for d in copies(c + 1, 1):
            d.start()
        for d in copies(c, 0):
            d.wait()
        compute(c, 0)

        @pl.when(i < NCH // 2 - 1)
        def _():
            for d in copies(c + 2, 0):
                d.start()

        for d in copies(c + 1, 1):
            d.wait()
        compute(c + 1, 1)
        return carry

    lax.fori_loop(0, NCH // 2, chunk_pair, 0)

    pltpu.sync_copy(outb, out_h.at[pl.ds(base, BPW)])
    pltpu.sync_copy(uid_v, uo_h.at[pl.ds(base, BPW)])
    pltpu.sync_copy(iid_v, io_h.at[pl.ds(base, BPW)])


def _emb_body(uid_h, iid_h, ue_h, ie_h, out_h,
              uid_v, iid_v, ueb, ieb, outb, sem0):
    wid = lax.axis_index("s") * NC + lax.axis_index("c")
    base = wid * BPW
    pltpu.sync_copy(uid_h.at[pl.ds(base, BPW)], uid_v)
    pltpu.sync_copy(iid_h.at[pl.ds(base, BPW)], iid_v)
    diags = _diags()
    iot = lax.iota(jnp.int32, L)

    ds = (
        pltpu.make_async_copy(ue_h.at[uid_v], ueb, sem0),
        pltpu.make_async_copy(ie_h.at[iid_v], ieb, sem0),
    )
    for d in ds:
        d.start()
    for d in ds:
        d.wait()

    def group(g, carry):
        rows = iot + g * L
        acc = jnp.zeros((L,), jnp.float32)
        for jb in range(EMB_DIM // L):
            cb = jnp.full((L,), jb * L, jnp.int32)
            for k in range(L):
                col = cb + diags[k]
                u = plsc.load_gather(ueb, [rows, col])
                v = plsc.load_gather(ieb, [rows, col])
                acc = acc + u * v
        outb[pl.ds(g * L, L)] = acc
        return carry

    lax.fori_loop(0, BPW // L, group, 0)

    pltpu.sync_copy(outb, out_h.at[pl.ds(base, BPW)])


def kernel(user_ids, item_ids, user_emb_w, item_emb_w, user_text_w,
           item_text_w, user_bias, item_bias, bias):
    mesh = plsc.VectorSubcoreMesh(core_axis_name="c", subcore_axis_name="s",
                                  num_cores=NC, num_subcores=NS)
    out_t = jax.ShapeDtypeStruct((B,), jnp.float32)
    id_t = jax.ShapeDtypeStruct((B,), jnp.int32)

    text_run = pl.kernel(
        _text_body,
        out_type=(out_t, id_t, id_t),
        mesh=mesh,
        compiler_params=pltpu.CompilerParams(use_tc_tiling_on_sc=True,
                                             needs_layout_passes=False),
        scratch_types=[
            pltpu.VMEM((BPW,), jnp.int32),
            pltpu.VMEM((BPW,), jnp.int32),
            pltpu.VMEM((2, C, BERT_DIM), jnp.float32),
            pltpu.VMEM((2, C, BERT_DIM), jnp.float32),
            pltpu.VMEM((2, C), jnp.float32),
            pltpu.VMEM((2, C), jnp.float32),
            pltpu.VMEM((BPW,), jnp.float32),
            pltpu.VMEM((L,), jnp.float32),
            pltpu.SemaphoreType.DMA,
            pltpu.SemaphoreType.DMA,
        ],
    )
    emb_run = pl.kernel(
        _emb_body,
        out_type=out_t,
        mesh=mesh,
        compiler_params=pltpu.CompilerParams(use_tc_tiling_on_sc=False,
                                             needs_layout_passes=False),
        scratch_types=[
            pltpu.VMEM((BPW,), jnp.int32),
            pltpu.VMEM((BPW,), jnp.int32),
            pltpu.VMEM((BPW, EMB_DIM), jnp.float32),
            pltpu.VMEM((BPW, EMB_DIM), jnp.float32),
            pltpu.VMEM((BPW,), jnp.float32),
            pltpu.SemaphoreType.DMA,
        ],
    )
    bias16 = jnp.broadcast_to(bias, (L,))
    out_text, uo, io = text_run(user_ids, item_ids, user_text_w,
                                item_text_w, user_bias, item_bias, bias16)
    out_emb = emb_run(uo, io, user_emb_w, item_emb_w)
    return (out_text + out_emb)[:, None]
